# Initial kernel scaffold; baseline (speedup 1.0000x reference)
import jax
import jax.numpy as jnp
from jax.experimental import pallas as pl


def kernel(x, conv_w, gamma, beta, alpha):
    raise NotImplementedError("write your pallas kernel here")


# speedup vs baseline: 2.5753x; 2.5753x over previous
import functools

import jax
import jax.numpy as jnp
from jax.experimental import pallas as pl
from jax.experimental.pallas import tpu as pltpu

_EPS = 1e-5
_C_IN = 3
_KH = _KW = 3
_K = _C_IN * _KH * _KW


def _round_up(a, b):
    return (a + b - 1) // b * b


def _pick_tile_m(mp_img, tile_m):
    q = mp_img // 128
    d_max = max(1, min(q, max(tile_m // 128, 1)))
    for d in range(d_max, 0, -1):
        if q % d == 0:
            return 128 * d
    return 128


def _pool3(p):
    pools = []
    for c in range(_C_IN):
        base = c * _KH * _KW
        m0 = jnp.maximum(p[base + 4:base + 5, :], p[base + 5:base + 6, :])
        m1 = jnp.maximum(p[base + 7:base + 8, :], p[base + 8:base + 9, :])
        pools.append(jnp.maximum(m0, m1))
    return jnp.concatenate(pools, axis=0).astype(jnp.float32)


def _stats_kernel(p_ref, w_ref, psum_ref, psq_ref):
    p = p_ref[0]
    conv = jnp.dot(w_ref[...], p,
                   preferred_element_type=jnp.float32)
    raw = jnp.concatenate([conv, _pool3(p)], axis=0)
    psum_ref[0] = jnp.sum(raw, axis=1, keepdims=True)
    psq_ref[0] = jnp.sum(raw * raw, axis=1, keepdims=True)


def _fused_out_kernel(p_ref, w_ref, scale_ref, shift_ref, alpha_ref, out_ref):
    p = p_ref[0]
    conv = jnp.dot(w_ref[...], p,
                   preferred_element_type=jnp.float32)
    raw = jnp.concatenate([conv, _pool3(p)], axis=0)
    y = raw * scale_ref[...] + shift_ref[...]
    out_ref[0] = jnp.where(y >= 0.0, y, alpha_ref[...] * y)


def _initial_block(x, conv_w, gamma, beta, alpha, *, tile_m=8192):
    N, c_in, H, W = x.shape
    assert c_in == _C_IN
    out_depth = conv_w.shape[0]
    c_total = out_depth + _C_IN
    Ho, Wo = H // 2, W // 2
    how = Ho * Wo
    M = N * how

    mp_img = _round_up(how, 128)
    TM = _pick_tile_m(mp_img, tile_m)
    tpi = mp_img // TM
    n_tiles = N * tpi

    xp = jnp.pad(x.astype(jnp.bfloat16), ((0, 0), (0, 0), (1, 1), (1, 1)))
    taps = [xp[:, :, kh:kh + 2 * Ho:2, kw:kw + 2 * Wo:2]
            for kh in range(_KH) for kw in range(_KW)]
    patches = jnp.stack(taps, axis=2).reshape(N, _K, how)
    if mp_img != how:
        patches = jnp.pad(patches, ((0, 0), (0, 0), (0, mp_img - how)))

    w_mat = conv_w.reshape(out_depth, _K).astype(jnp.bfloat16)

    psum, psq = pl.pallas_call(
        _stats_kernel,
        out_shape=(jax.ShapeDtypeStruct((n_tiles, c_total, 1), jnp.float32),
                   jax.ShapeDtypeStruct((n_tiles, c_total, 1), jnp.float32)),
        grid_spec=pltpu.PrefetchScalarGridSpec(
            num_scalar_prefetch=0,
            grid=(n_tiles,),
            in_specs=[pl.BlockSpec((1, _K, TM), lambda t: (t // tpi, 0, t % tpi)),
                      pl.BlockSpec((out_depth, _K), lambda t: (0, 0))],
            out_specs=[pl.BlockSpec((1, c_total, 1), lambda t: (t, 0, 0)),
                       pl.BlockSpec((1, c_total, 1), lambda t: (t, 0, 0))]),
        compiler_params=pltpu.CompilerParams(
            dimension_semantics=("parallel",),
            vmem_limit_bytes=64 * 1024 * 1024),
    )(patches, w_mat)

    ssum = jnp.sum(psum[:, :, 0], axis=0)
    ssq = jnp.sum(psq[:, :, 0], axis=0)
    mean = ssum / M
    var = jnp.maximum(ssq / M - mean * mean, 0.0)
    inv = jax.lax.rsqrt(var + _EPS)
    g = gamma.astype(jnp.float32)
    scale = (g * inv).reshape(c_total, 1)
    shift = (beta.astype(jnp.float32) - mean * g * inv).reshape(c_total, 1)
    alpha_col = jnp.broadcast_to(jnp.asarray(alpha, jnp.float32), (c_total, 1))

    y = pl.pallas_call(
        _fused_out_kernel,
        out_shape=jax.ShapeDtypeStruct((N, c_total, mp_img), jnp.float32),
        grid_spec=pltpu.PrefetchScalarGridSpec(
            num_scalar_prefetch=0,
            grid=(n_tiles,),
            in_specs=[pl.BlockSpec((1, _K, TM), lambda t: (t // tpi, 0, t % tpi)),
                      pl.BlockSpec((out_depth, _K), lambda t: (0, 0)),
                      pl.BlockSpec((c_total, 1), lambda t: (0, 0)),
                      pl.BlockSpec((c_total, 1), lambda t: (0, 0)),
                      pl.BlockSpec((c_total, 1), lambda t: (0, 0))],
            out_specs=pl.BlockSpec((1, c_total, TM),
                                   lambda t: (t // tpi, 0, t % tpi))),
        compiler_params=pltpu.CompilerParams(
            dimension_semantics=("parallel",),
            vmem_limit_bytes=64 * 1024 * 1024),
    )(patches, w_mat, scale, shift, alpha_col)

    return y[:, :, :how].reshape(N, c_total, Ho, Wo)


def kernel(x, conv_w, gamma, beta, alpha):
    return _initial_block(x, conv_w, gamma, beta, alpha)


# speedup vs baseline: 17.3516x; 6.7377x over previous
import functools

import jax
import jax.numpy as jnp
from jax.experimental import pallas as pl
from jax.experimental.pallas import tpu as pltpu

_EPS = 1e-5
_C_IN = 3
_KH = _KW = 3
_K = _C_IN * _KH * _KW


def _round_up(a, b):
    return (a + b - 1) // b * b


def _pick_tile_m(mp_img, tile_m):
    q = mp_img // 128
    d_max = max(1, min(q, max(tile_m // 128, 1)))
    for d in range(d_max, 0, -1):
        if q % d == 0:
            return 128 * d
    return 128


def _pool3(p):
    pools = []
    for c in range(_C_IN):
        base = c * _KH * _KW
        m0 = jnp.maximum(p[base + 4:base + 5, :], p[base + 5:base + 6, :])
        m1 = jnp.maximum(p[base + 7:base + 8, :], p[base + 8:base + 9, :])
        pools.append(jnp.maximum(m0, m1))
    return jnp.concatenate(pools, axis=0).astype(jnp.float32)


def _im2col_kernel(x_ref, t_ref, out_ref, *, ho, wo):
    t = t_ref[...]
    w_in = t.shape[0]
    for c in range(_C_IN):
        xe = x_ref[0, c, :, :w_in].astype(jnp.bfloat16)
        xo = x_ref[0, c, :, w_in:].astype(jnp.bfloat16)
        de = jnp.dot(xe, t, preferred_element_type=jnp.float32)
        do = jnp.dot(xo, t, preferred_element_type=jnp.float32)
        dm = jnp.concatenate(
            [jnp.zeros((1, 3 * wo), jnp.float32), do[:ho - 1]], axis=0)
        for kh, src in ((0, dm), (1, de), (2, do)):
            for kw in range(_KW):
                out_ref[0, c * _KH * _KW + kh * _KW + kw] = (
                    src[:, kw * wo:(kw + 1) * wo].astype(jnp.bfloat16))


def _im2col_pallas(x, ho, wo):
    n = x.shape[0]
    w_in = x.shape[3]
    x4 = x.reshape(n, _C_IN, ho, 2 * w_in)
    j = jax.lax.broadcasted_iota(jnp.int32, (w_in, _KW * wo), 0)
    col = jax.lax.broadcasted_iota(jnp.int32, (w_in, _KW * wo), 1)
    sel = 2 * (col % wo) + col // wo - 1
    t = (j == sel).astype(jnp.bfloat16)

    kern = functools.partial(_im2col_kernel, ho=ho, wo=wo)
    patches = pl.pallas_call(
        kern,
        out_shape=jax.ShapeDtypeStruct((n, _K, ho, wo), jnp.bfloat16),
        grid_spec=pltpu.PrefetchScalarGridSpec(
            num_scalar_prefetch=0,
            grid=(n,),
            in_specs=[pl.BlockSpec((1, _C_IN, ho, 2 * w_in),
                                   lambda i: (i, 0, 0, 0)),
                      pl.BlockSpec((w_in, _KW * wo), lambda i: (0, 0))],
            out_specs=pl.BlockSpec((1, _K, ho, wo), lambda i: (i, 0, 0, 0))),
        compiler_params=pltpu.CompilerParams(
            dimension_semantics=("parallel",),
            vmem_limit_bytes=64 * 1024 * 1024),
    )(x4, t)
    return patches.reshape(n, _K, ho * wo)


def _stats_kernel(p_ref, w_ref, psum_ref, psq_ref):
    p = p_ref[0]
    conv = jnp.dot(w_ref[...], p,
                   preferred_element_type=jnp.float32)
    raw = jnp.concatenate([conv, _pool3(p)], axis=0)
    psum_ref[0] = jnp.sum(raw, axis=1, keepdims=True)
    psq_ref[0] = jnp.sum(raw * raw, axis=1, keepdims=True)


def _fused_out_kernel(p_ref, w_ref, scale_ref, shift_ref, alpha_ref, out_ref):
    p = p_ref[0]
    conv = jnp.dot(w_ref[...], p,
                   preferred_element_type=jnp.float32)
    raw = jnp.concatenate([conv, _pool3(p)], axis=0)
    y = raw * scale_ref[...] + shift_ref[...]
    out_ref[0] = jnp.where(y >= 0.0, y, alpha_ref[...] * y)


def _initial_block(x, conv_w, gamma, beta, alpha, *, tile_m=8192):
    N, c_in, H, W = x.shape
    assert c_in == _C_IN
    out_depth = conv_w.shape[0]
    c_total = out_depth + _C_IN
    Ho, Wo = H // 2, W // 2
    how = Ho * Wo
    M = N * how

    mp_img = _round_up(how, 128)
    TM = _pick_tile_m(mp_img, tile_m)
    tpi = mp_img // TM
    n_tiles = N * tpi

    if Wo % 128 == 0 and mp_img == how:
        patches = _im2col_pallas(x, Ho, Wo)
    else:
        xpd = jnp.pad(x.astype(jnp.bfloat16), ((0, 0), (0, 0), (1, 1), (1, 1)))
        taps = [xpd[:, :, kh:kh + 2 * Ho:2, kw:kw + 2 * Wo:2]
                for kh in range(_KH) for kw in range(_KW)]
        patches = jnp.stack(taps, axis=2).reshape(N, _K, how)
        if mp_img != how:
            patches = jnp.pad(patches, ((0, 0), (0, 0), (0, mp_img - how)))

    w_mat = conv_w.reshape(out_depth, _K).astype(jnp.bfloat16)

    psum, psq = pl.pallas_call(
        _stats_kernel,
        out_shape=(jax.ShapeDtypeStruct((n_tiles, c_total, 1), jnp.float32),
                   jax.ShapeDtypeStruct((n_tiles, c_total, 1), jnp.float32)),
        grid_spec=pltpu.PrefetchScalarGridSpec(
            num_scalar_prefetch=0,
            grid=(n_tiles,),
            in_specs=[pl.BlockSpec((1, _K, TM), lambda t: (t // tpi, 0, t % tpi)),
                      pl.BlockSpec((out_depth, _K), lambda t: (0, 0))],
            out_specs=[pl.BlockSpec((1, c_total, 1), lambda t: (t, 0, 0)),
                       pl.BlockSpec((1, c_total, 1), lambda t: (t, 0, 0))]),
        compiler_params=pltpu.CompilerParams(
            dimension_semantics=("parallel",),
            vmem_limit_bytes=64 * 1024 * 1024),
    )(patches, w_mat)

    ssum = jnp.sum(psum[:, :, 0], axis=0)
    ssq = jnp.sum(psq[:, :, 0], axis=0)
    mean = ssum / M
    var = jnp.maximum(ssq / M - mean * mean, 0.0)
    inv = jax.lax.rsqrt(var + _EPS)
    g = gamma.astype(jnp.float32)
    scale = (g * inv).reshape(c_total, 1)
    shift = (beta.astype(jnp.float32) - mean * g * inv).reshape(c_total, 1)
    alpha_col = jnp.broadcast_to(jnp.asarray(alpha, jnp.float32), (c_total, 1))

    y = pl.pallas_call(
        _fused_out_kernel,
        out_shape=jax.ShapeDtypeStruct((N, c_total, mp_img), jnp.float32),
        grid_spec=pltpu.PrefetchScalarGridSpec(
            num_scalar_prefetch=0,
            grid=(n_tiles,),
            in_specs=[pl.BlockSpec((1, _K, TM), lambda t: (t // tpi, 0, t % tpi)),
                      pl.BlockSpec((out_depth, _K), lambda t: (0, 0)),
                      pl.BlockSpec((c_total, 1), lambda t: (0, 0)),
                      pl.BlockSpec((c_total, 1), lambda t: (0, 0)),
                      pl.BlockSpec((c_total, 1), lambda t: (0, 0))],
            out_specs=pl.BlockSpec((1, c_total, TM),
                                   lambda t: (t // tpi, 0, t % tpi))),
        compiler_params=pltpu.CompilerParams(
            dimension_semantics=("parallel",),
            vmem_limit_bytes=64 * 1024 * 1024),
    )(patches, w_mat, scale, shift, alpha_col)

    return y[:, :, :how].reshape(N, c_total, Ho, Wo)


def kernel(x, conv_w, gamma, beta, alpha):
    return _initial_block(x, conv_w, gamma, beta, alpha)


# speedup vs baseline: 26.5146x; 1.5281x over previous
import functools

import jax
import jax.numpy as jnp
from jax.experimental import pallas as pl
from jax.experimental.pallas import tpu as pltpu

_EPS = 1e-5
_C_IN = 3
_KH = _KW = 3
_K = _C_IN * _KH * _KW
_G = 8


def _im2col_kernel(x_ref, t_ref, out_ref, *, ho, wo):
    t = t_ref[...]
    w_in = t.shape[0]
    for c in range(_C_IN):
        xe = x_ref[0, c, :, :w_in].astype(jnp.bfloat16)
        xo = x_ref[0, c, :, w_in:].astype(jnp.bfloat16)
        de = jnp.dot(xe, t, preferred_element_type=jnp.float32)
        do = jnp.dot(xo, t, preferred_element_type=jnp.float32)
        dm = jnp.concatenate(
            [jnp.zeros((1, _KW * wo), jnp.float32), do[:ho - 1]], axis=0)
        for kh, src in ((0, dm), (1, de), (2, do)):
            for kw in range(_KW):
                out_ref[0, c * _KH * _KW + kh * _KW + kw] = (
                    src[:, kw * wo:(kw + 1) * wo].astype(jnp.bfloat16))


def _im2col_pallas(x, ho, wo):
    n, _, h, w_in = x.shape
    x4 = x.reshape(n, _C_IN, ho, 2 * w_in)
    j = jax.lax.broadcasted_iota(jnp.int32, (w_in, _KW * wo), 0)
    col = jax.lax.broadcasted_iota(jnp.int32, (w_in, _KW * wo), 1)
    sel = 2 * (col % wo) + col // wo - 1
    t = (j == sel).astype(jnp.bfloat16)

    kern = functools.partial(_im2col_kernel, ho=ho, wo=wo)
    return pl.pallas_call(
        kern,
        out_shape=jax.ShapeDtypeStruct((n, _K, ho, wo), jnp.bfloat16),
        grid_spec=pltpu.PrefetchScalarGridSpec(
            num_scalar_prefetch=0,
            grid=(n,),
            in_specs=[pl.BlockSpec((1, _C_IN, ho, 2 * w_in),
                                   lambda i: (i, 0, 0, 0)),
                      pl.BlockSpec((w_in, _KW * wo), lambda i: (0, 0))],
            out_specs=pl.BlockSpec((1, _K, ho, wo), lambda i: (i, 0, 0, 0))),
        compiler_params=pltpu.CompilerParams(
            dimension_semantics=("parallel",),
            vmem_limit_bytes=64 * 1024 * 1024),
    )(x4, t)


def _conv_groups(p4, l_ref, rb, wo):
    l = l_ref[...]
    zpad = jnp.zeros((2 * 128 - _K * _G, wo), jnp.bfloat16)
    outs = []
    for g in range(rb // _G):
        rhs = p4[:, g * _G:(g + 1) * _G, :].reshape(_K * _G, wo)
        rhs = jnp.concatenate([rhs, zpad], axis=0)
        stk = jnp.dot(l, rhs, preferred_element_type=jnp.float32)
        outs.append(stk.reshape(16, _G, wo))
    return jnp.concatenate(outs, axis=1)


def _pool3(p4):
    pools = []
    for c in range(_C_IN):
        b = c * _KH * _KW
        m0 = jnp.maximum(p4[b + 4], p4[b + 5])
        m1 = jnp.maximum(p4[b + 7], p4[b + 8])
        pools.append(jnp.maximum(m0, m1)[None])
    return jnp.concatenate(pools, axis=0).astype(jnp.float32)


def _stats_kernel(p_ref, l_ref, psum_ref, psq_ref, *, rb, wo, out_depth):
    p4 = p_ref[0]
    conv = _conv_groups(p4, l_ref, rb, wo)
    raw = jnp.concatenate([conv[:out_depth], _pool3(p4)], axis=0)
    psum_ref[0] = jnp.sum(jnp.sum(raw, axis=2), axis=1, keepdims=True)
    psq_ref[0] = jnp.sum(jnp.sum(raw * raw, axis=2), axis=1, keepdims=True)


def _out_kernel(p_ref, l_ref, scale_ref, shift_ref, alpha_ref, out_ref,
                *, rb, wo, out_depth):
    p4 = p_ref[0]
    conv = _conv_groups(p4, l_ref, rb, wo)
    raw = jnp.concatenate([conv[:out_depth], _pool3(p4)], axis=0)
    y = raw * scale_ref[...][:, :, None] + shift_ref[...][:, :, None]
    out_ref[0] = jnp.where(y >= 0.0, y, alpha_ref[...][:, :, None] * y)


def _initial_block(x, conv_w, gamma, beta, alpha):
    N, c_in, H, W = x.shape
    assert c_in == _C_IN
    out_depth = conv_w.shape[0]
    c_total = out_depth + _C_IN
    Ho, Wo = H // 2, W // 2
    M = N * Ho * Wo

    rb = 32 if Ho % 32 == 0 else _G
    assert Wo % 128 == 0 and Ho % rb == 0, "unsupported shape"
    tpi = Ho // rb
    n_tiles = N * tpi

    patches = _im2col_pallas(x, Ho, Wo)

    w_mat = jnp.pad(conv_w.reshape(out_depth, _K), ((0, c_total - out_depth),
                                                    (0, 0)))
    l_mat = jnp.kron(w_mat, jnp.eye(_G, dtype=w_mat.dtype))
    l_mat = jnp.pad(l_mat, ((0, 0), (0, 2 * 128 - _K * _G)))
    l_mat = l_mat.astype(jnp.bfloat16)

    kern_a = functools.partial(_stats_kernel, rb=rb, wo=Wo, out_depth=out_depth)
    psum, psq = pl.pallas_call(
        kern_a,
        out_shape=(jax.ShapeDtypeStruct((n_tiles, c_total, 1), jnp.float32),
                   jax.ShapeDtypeStruct((n_tiles, c_total, 1), jnp.float32)),
        grid_spec=pltpu.PrefetchScalarGridSpec(
            num_scalar_prefetch=0,
            grid=(n_tiles,),
            in_specs=[pl.BlockSpec((1, _K, rb, Wo),
                                   lambda i: (i // tpi, 0, i % tpi, 0)),
                      pl.BlockSpec((c_total * _G, 2 * 128),
                                   lambda i: (0, 0))],
            out_specs=[pl.BlockSpec((1, c_total, 1), lambda i: (i, 0, 0)),
                       pl.BlockSpec((1, c_total, 1), lambda i: (i, 0, 0))]),
        compiler_params=pltpu.CompilerParams(
            dimension_semantics=("parallel",),
            vmem_limit_bytes=64 * 1024 * 1024),
    )(patches, l_mat)

    ssum = jnp.sum(psum[:, :, 0], axis=0)
    ssq = jnp.sum(psq[:, :, 0], axis=0)
    mean = ssum / M
    var = jnp.maximum(ssq / M - mean * mean, 0.0)
    inv = jax.lax.rsqrt(var + _EPS)
    g = gamma.astype(jnp.float32)
    scale = (g * inv).reshape(c_total, 1)
    shift = (beta.astype(jnp.float32) - mean * g * inv).reshape(c_total, 1)
    alpha_col = jnp.broadcast_to(jnp.asarray(alpha, jnp.float32), (c_total, 1))

    kern_b = functools.partial(_out_kernel, rb=rb, wo=Wo, out_depth=out_depth)
    y = pl.pallas_call(
        kern_b,
        out_shape=jax.ShapeDtypeStruct((N, c_total, Ho, Wo), jnp.float32),
        grid_spec=pltpu.PrefetchScalarGridSpec(
            num_scalar_prefetch=0,
            grid=(n_tiles,),
            in_specs=[pl.BlockSpec((1, _K, rb, Wo),
                                   lambda i: (i // tpi, 0, i % tpi, 0)),
                      pl.BlockSpec((c_total * _G, 2 * 128),
                                   lambda i: (0, 0)),
                      pl.BlockSpec((c_total, 1), lambda i: (0, 0)),
                      pl.BlockSpec((c_total, 1), lambda i: (0, 0)),
                      pl.BlockSpec((c_total, 1), lambda i: (0, 0))],
            out_specs=pl.BlockSpec((1, c_total, rb, Wo),
                                   lambda i: (i // tpi, 0, i % tpi, 0))),
        compiler_params=pltpu.CompilerParams(
            dimension_semantics=("parallel",),
            vmem_limit_bytes=64 * 1024 * 1024),
    )(patches, l_mat, scale, shift, alpha_col)

    return y


def kernel(x, conv_w, gamma, beta, alpha):
    return _initial_block(x, conv_w, gamma, beta, alpha)


# speedup vs baseline: 34.2893x; 1.2932x over previous
import functools

import jax
import jax.numpy as jnp
from jax.experimental import pallas as pl
from jax.experimental.pallas import tpu as pltpu

_EPS = 1e-5
_C_IN = 3
_KH = _KW = 3
_K = _C_IN * _KH * _KW
_G = 8


def _im2col_stats_kernel(x_ref, t_ref, l_ref, out_ref, psum_ref, psq_ref,
                         *, ho, wo, out_depth):
    t = t_ref[...]
    w_in = t.shape[0]
    pool_parts = []
    for c in range(_C_IN):
        xe = x_ref[0, c, :, :w_in].astype(jnp.bfloat16)
        xo = x_ref[0, c, :, w_in:].astype(jnp.bfloat16)
        de = jnp.dot(xe, t, preferred_element_type=jnp.float32)
        do = jnp.dot(xo, t, preferred_element_type=jnp.float32)
        dm = jnp.concatenate(
            [jnp.zeros((1, _KW * wo), jnp.float32), do[:ho - 1]], axis=0)
        for kh, src in ((0, dm), (1, de), (2, do)):
            for kw in range(_KW):
                out_ref[0, c * _KH * _KW + kh * _KW + kw] = (
                    src[:, kw * wo:(kw + 1) * wo].astype(jnp.bfloat16))
        pool_c = jnp.maximum(jnp.maximum(de[:, wo:2 * wo], de[:, 2 * wo:]),
                             jnp.maximum(do[:, wo:2 * wo], do[:, 2 * wo:]))
        pool_parts.append(pool_c)

    l = l_ref[...]
    zpad = jnp.zeros((2 * 128 - _K * _G, wo), jnp.bfloat16)
    s_conv = jnp.zeros((16, wo), jnp.float32)
    q_conv = jnp.zeros((16, wo), jnp.float32)
    for g in range(ho // _G):
        rhs = out_ref[0, :, pl.ds(g * _G, _G), :]
        rhs = jnp.concatenate([rhs.reshape(_K * _G, wo), zpad], axis=0)
        stk = jnp.dot(l, rhs, preferred_element_type=jnp.float32)
        rs = stk.reshape(16, _G, wo)
        s_conv = s_conv + jnp.sum(rs, axis=1)
        q_conv = q_conv + jnp.sum(rs * rs, axis=1)
    s_col = jnp.sum(s_conv, axis=1, keepdims=True)
    q_col = jnp.sum(q_conv, axis=1, keepdims=True)
    pool_s = jnp.concatenate(
        [jnp.sum(jnp.sum(p, axis=0, keepdims=True), axis=1, keepdims=True)
         for p in pool_parts], axis=0)
    pool_q = jnp.concatenate(
        [jnp.sum(jnp.sum(p * p, axis=0, keepdims=True), axis=1, keepdims=True)
         for p in pool_parts], axis=0)
    psum_ref[0] = jnp.concatenate([s_col[:out_depth], pool_s], axis=0)
    psq_ref[0] = jnp.concatenate([q_col[:out_depth], pool_q], axis=0)


def _im2col_pallas(x, l_mat, ho, wo, out_depth, c_total):
    n, _, h, w_in = x.shape
    x4 = x.reshape(n, _C_IN, ho, 2 * w_in)
    j = jax.lax.broadcasted_iota(jnp.int32, (w_in, _KW * wo), 0)
    col = jax.lax.broadcasted_iota(jnp.int32, (w_in, _KW * wo), 1)
    sel = 2 * (col % wo) + col // wo - 1
    t = (j == sel).astype(jnp.bfloat16)

    kern = functools.partial(_im2col_stats_kernel, ho=ho, wo=wo,
                             out_depth=out_depth)
    return pl.pallas_call(
        kern,
        out_shape=(jax.ShapeDtypeStruct((n, _K, ho, wo), jnp.bfloat16),
                   jax.ShapeDtypeStruct((n, c_total, 1), jnp.float32),
                   jax.ShapeDtypeStruct((n, c_total, 1), jnp.float32)),
        grid_spec=pltpu.PrefetchScalarGridSpec(
            num_scalar_prefetch=0,
            grid=(n,),
            in_specs=[pl.BlockSpec((1, _C_IN, ho, 2 * w_in),
                                   lambda i: (i, 0, 0, 0)),
                      pl.BlockSpec((w_in, _KW * wo), lambda i: (0, 0)),
                      pl.BlockSpec((c_total * _G, 2 * 128),
                                   lambda i: (0, 0))],
            out_specs=[pl.BlockSpec((1, _K, ho, wo), lambda i: (i, 0, 0, 0)),
                       pl.BlockSpec((1, c_total, 1), lambda i: (i, 0, 0)),
                       pl.BlockSpec((1, c_total, 1), lambda i: (i, 0, 0))]),
        compiler_params=pltpu.CompilerParams(
            dimension_semantics=("parallel",),
            vmem_limit_bytes=64 * 1024 * 1024),
    )(x4, t, l_mat)


def _conv_groups(p4, l_ref, rb, wo):
    l = l_ref[...]
    zpad = jnp.zeros((2 * 128 - _K * _G, wo), jnp.bfloat16)
    outs = []
    for g in range(rb // _G):
        rhs = p4[:, g * _G:(g + 1) * _G, :].reshape(_K * _G, wo)
        rhs = jnp.concatenate([rhs, zpad], axis=0)
        stk = jnp.dot(l, rhs, preferred_element_type=jnp.float32)
        outs.append(stk.reshape(16, _G, wo))
    return jnp.concatenate(outs, axis=1)


def _pool3(p4):
    pools = []
    for c in range(_C_IN):
        b = c * _KH * _KW
        m0 = jnp.maximum(p4[b + 4], p4[b + 5])
        m1 = jnp.maximum(p4[b + 7], p4[b + 8])
        pools.append(jnp.maximum(m0, m1)[None])
    return jnp.concatenate(pools, axis=0).astype(jnp.float32)


def _out_kernel(p_ref, l_ref, scale_ref, shift_ref, alpha_ref, out_ref,
                *, rb, wo, out_depth):
    p4 = p_ref[0]
    conv = _conv_groups(p4, l_ref, rb, wo)
    raw = jnp.concatenate([conv[:out_depth], _pool3(p4)], axis=0)
    y = raw * scale_ref[...][:, :, None] + shift_ref[...][:, :, None]
    out_ref[0] = jnp.where(y >= 0.0, y, alpha_ref[...][:, :, None] * y)


def _initial_block(x, conv_w, gamma, beta, alpha):
    N, c_in, H, W = x.shape
    assert c_in == _C_IN
    out_depth = conv_w.shape[0]
    c_total = out_depth + _C_IN
    Ho, Wo = H // 2, W // 2
    M = N * Ho * Wo

    rb = 32 if Ho % 32 == 0 else _G
    assert Wo % 128 == 0 and Ho % rb == 0, "unsupported shape"
    tpi = Ho // rb
    n_tiles = N * tpi

    w_mat = jnp.pad(conv_w.reshape(out_depth, _K), ((0, c_total - out_depth),
                                                    (0, 0)))
    l_mat = jnp.kron(w_mat, jnp.eye(_G, dtype=w_mat.dtype))
    l_mat = jnp.pad(l_mat, ((0, 0), (0, 2 * 128 - _K * _G)))
    l_mat = l_mat.astype(jnp.bfloat16)

    patches, psum, psq = _im2col_pallas(x, l_mat, Ho, Wo, out_depth, c_total)

    ssum = jnp.sum(psum[:, :, 0], axis=0)
    ssq = jnp.sum(psq[:, :, 0], axis=0)
    mean = ssum / M
    var = jnp.maximum(ssq / M - mean * mean, 0.0)
    inv = jax.lax.rsqrt(var + _EPS)
    g = gamma.astype(jnp.float32)
    scale = (g * inv).reshape(c_total, 1)
    shift = (beta.astype(jnp.float32) - mean * g * inv).reshape(c_total, 1)
    alpha_col = jnp.broadcast_to(jnp.asarray(alpha, jnp.float32), (c_total, 1))

    kern_b = functools.partial(_out_kernel, rb=rb, wo=Wo, out_depth=out_depth)
    y = pl.pallas_call(
        kern_b,
        out_shape=jax.ShapeDtypeStruct((N, c_total, Ho, Wo), jnp.float32),
        grid_spec=pltpu.PrefetchScalarGridSpec(
            num_scalar_prefetch=0,
            grid=(n_tiles,),
            in_specs=[pl.BlockSpec((1, _K, rb, Wo),
                                   lambda i: (i // tpi, 0, i % tpi, 0)),
                      pl.BlockSpec((c_total * _G, 2 * 128),
                                   lambda i: (0, 0)),
                      pl.BlockSpec((c_total, 1), lambda i: (0, 0)),
                      pl.BlockSpec((c_total, 1), lambda i: (0, 0)),
                      pl.BlockSpec((c_total, 1), lambda i: (0, 0))],
            out_specs=pl.BlockSpec((1, c_total, rb, Wo),
                                   lambda i: (i // tpi, 0, i % tpi, 0))),
        compiler_params=pltpu.CompilerParams(
            dimension_semantics=("parallel",),
            vmem_limit_bytes=64 * 1024 * 1024),
    )(patches, l_mat, scale, shift, alpha_col)

    return y


def kernel(x, conv_w, gamma, beta, alpha):
    return _initial_block(x, conv_w, gamma, beta, alpha)


# speedup vs baseline: 44.6689x; 1.3027x over previous
import functools

import jax
import jax.numpy as jnp
from jax.experimental import pallas as pl
from jax.experimental.pallas import tpu as pltpu

_EPS = 1e-5
_C_IN = 3
_KH = _KW = 3
_K = _C_IN * _KH * _KW
_G = 8


def _im2col_stats_kernel(x_ref, t_ref, l_ref, out_ref, psum_ref, psq_ref,
                         *, ho, wo, out_depth):
    t = t_ref[...]
    w_in = t.shape[0]
    pool_parts = []
    for c in range(_C_IN):
        xe = x_ref[0, c, :, :w_in].astype(jnp.bfloat16)
        xo = x_ref[0, c, :, w_in:].astype(jnp.bfloat16)
        de = jnp.dot(xe, t, preferred_element_type=jnp.float32)
        do = jnp.dot(xo, t, preferred_element_type=jnp.float32)
        dm = jnp.concatenate(
            [jnp.zeros((1, _KW * wo), jnp.float32), do[:ho - 1]], axis=0)
        for kh, src in ((0, dm), (1, de), (2, do)):
            for kw in range(_KW):
                out_ref[0, c * _KH * _KW + kh * _KW + kw] = (
                    src[:, kw * wo:(kw + 1) * wo].astype(jnp.bfloat16))
        pool_c = jnp.maximum(jnp.maximum(de[:, wo:2 * wo], de[:, 2 * wo:]),
                             jnp.maximum(do[:, wo:2 * wo], do[:, 2 * wo:]))
        pool_parts.append(pool_c)

    l = l_ref[...]
    zpad = jnp.zeros((2 * 128 - _K * _G, wo), jnp.bfloat16)
    s_conv = jnp.zeros((16, wo), jnp.float32)
    q_conv = jnp.zeros((16, wo), jnp.float32)
    for g in range(ho // _G):
        rhs = out_ref[0, :, pl.ds(g * _G, _G), :]
        rhs = jnp.concatenate([rhs.reshape(_K * _G, wo), zpad], axis=0)
        stk = jnp.dot(l, rhs, preferred_element_type=jnp.float32)
        rs = stk.reshape(16, _G, wo)
        s_conv = s_conv + jnp.sum(rs, axis=1)
        q_conv = q_conv + jnp.sum(rs * rs, axis=1)
    s_col = jnp.sum(s_conv, axis=1, keepdims=True)
    q_col = jnp.sum(q_conv, axis=1, keepdims=True)
    pool_s = jnp.concatenate(
        [jnp.sum(jnp.sum(p, axis=0, keepdims=True), axis=1, keepdims=True)
         for p in pool_parts], axis=0)
    pool_q = jnp.concatenate(
        [jnp.sum(jnp.sum(p * p, axis=0, keepdims=True), axis=1, keepdims=True)
         for p in pool_parts], axis=0)
    psum_ref[0] = jnp.concatenate([s_col[:out_depth], pool_s], axis=0)
    psq_ref[0] = jnp.concatenate([q_col[:out_depth], pool_q], axis=0)


def _im2col_pallas(x, l_mat, ho, wo, out_depth, c_total):
    n, _, h, w_in = x.shape
    x4 = x.reshape(n, _C_IN, ho, 2 * w_in)
    j = jax.lax.broadcasted_iota(jnp.int32, (w_in, _KW * wo), 0)
    col = jax.lax.broadcasted_iota(jnp.int32, (w_in, _KW * wo), 1)
    sel = 2 * (col % wo) + col // wo - 1
    t = (j == sel).astype(jnp.bfloat16)

    kern = functools.partial(_im2col_stats_kernel, ho=ho, wo=wo,
                             out_depth=out_depth)
    return pl.pallas_call(
        kern,
        out_shape=(jax.ShapeDtypeStruct((n, _K, ho, wo), jnp.bfloat16),
                   jax.ShapeDtypeStruct((n, c_total, 1), jnp.float32),
                   jax.ShapeDtypeStruct((n, c_total, 1), jnp.float32)),
        grid_spec=pltpu.PrefetchScalarGridSpec(
            num_scalar_prefetch=0,
            grid=(n,),
            in_specs=[pl.BlockSpec((1, _C_IN, ho, 2 * w_in),
                                   lambda i: (i, 0, 0, 0)),
                      pl.BlockSpec((w_in, _KW * wo), lambda i: (0, 0)),
                      pl.BlockSpec((c_total * _G, 2 * 128),
                                   lambda i: (0, 0))],
            out_specs=[pl.BlockSpec((1, _K, ho, wo), lambda i: (i, 0, 0, 0)),
                       pl.BlockSpec((1, c_total, 1), lambda i: (i, 0, 0)),
                       pl.BlockSpec((1, c_total, 1), lambda i: (i, 0, 0))]),
        compiler_params=pltpu.CompilerParams(
            dimension_semantics=("parallel",),
            vmem_limit_bytes=64 * 1024 * 1024),
    )(x4, t, l_mat)


def _conv_groups(p4, l_ref, rb, wo):
    l = l_ref[...]
    zpad = jnp.zeros((2 * 128 - _K * _G, wo), jnp.bfloat16)
    outs = []
    for g in range(rb // _G):
        rhs = p4[:, g * _G:(g + 1) * _G, :].reshape(_K * _G, wo)
        rhs = jnp.concatenate([rhs, zpad], axis=0)
        stk = jnp.dot(l, rhs, preferred_element_type=jnp.float32)
        outs.append(stk.reshape(16, _G, wo))
    return jnp.concatenate(outs, axis=1)


def _pool3(p4):
    pools = []
    for c in range(_C_IN):
        b = c * _KH * _KW
        m0 = jnp.maximum(p4[b + 4], p4[b + 5])
        m1 = jnp.maximum(p4[b + 7], p4[b + 8])
        pools.append(jnp.maximum(m0, m1)[None])
    return jnp.concatenate(pools, axis=0).astype(jnp.float32)


def _out_kernel(p_ref, l_ref, scale_ref, shift_ref, alpha_ref, out_ref,
                *, rb, wo, out_depth):
    p4 = p_ref[0]
    conv = _conv_groups(p4, l_ref, rb, wo)
    raw = jnp.concatenate([conv[:out_depth], _pool3(p4)], axis=0)
    y = raw * scale_ref[...][:, :, None] + shift_ref[...][:, :, None]
    out_ref[0] = jnp.where(y >= 0.0, y, alpha_ref[...][:, :, None] * y)


def _initial_block(x, conv_w, gamma, beta, alpha):
    N, c_in, H, W = x.shape
    assert c_in == _C_IN
    out_depth = conv_w.shape[0]
    c_total = out_depth + _C_IN
    Ho, Wo = H // 2, W // 2
    M = N * Ho * Wo

    rb = _G
    for cand_rb in (128, 64, 32, 16):
        if Ho % cand_rb == 0:
            rb = cand_rb
            break
    assert Wo % 128 == 0 and Ho % rb == 0, "unsupported shape"
    tpi = Ho // rb
    n_tiles = N * tpi

    w_mat = jnp.pad(conv_w.reshape(out_depth, _K), ((0, c_total - out_depth),
                                                    (0, 0)))
    l_mat = jnp.kron(w_mat, jnp.eye(_G, dtype=w_mat.dtype))
    l_mat = jnp.pad(l_mat, ((0, 0), (0, 2 * 128 - _K * _G)))
    l_mat = l_mat.astype(jnp.bfloat16)

    patches, psum, psq = _im2col_pallas(x, l_mat, Ho, Wo, out_depth, c_total)

    ssum = jnp.sum(psum[:, :, 0], axis=0)
    ssq = jnp.sum(psq[:, :, 0], axis=0)
    mean = ssum / M
    var = jnp.maximum(ssq / M - mean * mean, 0.0)
    inv = jax.lax.rsqrt(var + _EPS)
    g = gamma.astype(jnp.float32)
    scale = (g * inv).reshape(c_total, 1)
    shift = (beta.astype(jnp.float32) - mean * g * inv).reshape(c_total, 1)
    alpha_col = jnp.broadcast_to(jnp.asarray(alpha, jnp.float32), (c_total, 1))

    kern_b = functools.partial(_out_kernel, rb=rb, wo=Wo, out_depth=out_depth)
    y = pl.pallas_call(
        kern_b,
        out_shape=jax.ShapeDtypeStruct((N, c_total, Ho, Wo), jnp.float32),
        grid_spec=pltpu.PrefetchScalarGridSpec(
            num_scalar_prefetch=0,
            grid=(n_tiles,),
            in_specs=[pl.BlockSpec((1, _K, rb, Wo),
                                   lambda i: (i // tpi, 0, i % tpi, 0)),
                      pl.BlockSpec((c_total * _G, 2 * 128),
                                   lambda i: (0, 0)),
                      pl.BlockSpec((c_total, 1), lambda i: (0, 0)),
                      pl.BlockSpec((c_total, 1), lambda i: (0, 0)),
                      pl.BlockSpec((c_total, 1), lambda i: (0, 0))],
            out_specs=pl.BlockSpec((1, c_total, rb, Wo),
                                   lambda i: (i // tpi, 0, i % tpi, 0))),
        compiler_params=pltpu.CompilerParams(
            dimension_semantics=("parallel",),
            vmem_limit_bytes=64 * 1024 * 1024),
    )(patches, l_mat, scale, shift, alpha_col)

    return y


def kernel(x, conv_w, gamma, beta, alpha):
    return _initial_block(x, conv_w, gamma, beta, alpha)


# speedup vs baseline: 46.6971x; 1.0454x over previous
import functools

import jax
import jax.numpy as jnp
from jax.experimental import pallas as pl
from jax.experimental.pallas import tpu as pltpu

_EPS = 1e-5
_C_IN = 3
_KH = _KW = 3
_K = _C_IN * _KH * _KW
_G = 8


def _im2col_stats_kernel(x_ref, t_ref, l_ref, out_ref, psum_ref, psq_ref,
                         *, ho, wo, out_depth):
    t = t_ref[...]
    w_in = t.shape[0]
    pool_parts = []
    for c in range(_C_IN):
        xe = x_ref[0, c, :, :w_in].astype(jnp.bfloat16)
        xo = x_ref[0, c, :, w_in:].astype(jnp.bfloat16)
        de = jnp.dot(xe, t, preferred_element_type=jnp.float32)
        do = jnp.dot(xo, t, preferred_element_type=jnp.float32)
        dm = jnp.concatenate(
            [jnp.zeros((1, _KW * wo), jnp.float32), do[:ho - 1]], axis=0)
        for kh, src in ((0, dm), (1, de), (2, do)):
            for kw in range(_KW):
                out_ref[0, c * _KH * _KW + kh * _KW + kw] = (
                    src[:, kw * wo:(kw + 1) * wo].astype(jnp.bfloat16))
        pool_c = jnp.maximum(jnp.maximum(de[:, wo:2 * wo], de[:, 2 * wo:]),
                             jnp.maximum(do[:, wo:2 * wo], do[:, 2 * wo:]))
        pool_parts.append(pool_c)

    l = l_ref[...]
    zpad = jnp.zeros((2 * 128 - _K * _G, wo), jnp.bfloat16)
    s_conv = jnp.zeros((16, wo), jnp.float32)
    q_conv = jnp.zeros((16, wo), jnp.float32)
    for g in range(ho // _G):
        rhs = out_ref[0, :, pl.ds(g * _G, _G), :]
        rhs = jnp.concatenate([rhs.reshape(_K * _G, wo), zpad], axis=0)
        stk = jnp.dot(l, rhs, preferred_element_type=jnp.float32)
        rs = stk.reshape(16, _G, wo)
        s_conv = s_conv + jnp.sum(rs, axis=1)
        q_conv = q_conv + jnp.sum(rs * rs, axis=1)
    s_col = jnp.sum(s_conv, axis=1, keepdims=True)
    q_col = jnp.sum(q_conv, axis=1, keepdims=True)
    pool_s = jnp.concatenate(
        [jnp.sum(jnp.sum(p, axis=0, keepdims=True), axis=1, keepdims=True)
         for p in pool_parts], axis=0)
    pool_q = jnp.concatenate(
        [jnp.sum(jnp.sum(p * p, axis=0, keepdims=True), axis=1, keepdims=True)
         for p in pool_parts], axis=0)
    psum_ref[0] = jnp.concatenate([s_col[:out_depth], pool_s], axis=0)
    psq_ref[0] = jnp.concatenate([q_col[:out_depth], pool_q], axis=0)


def _im2col_pallas(x, l_mat, ho, wo, out_depth, c_total):
    n, _, h, w_in = x.shape
    x4 = x.reshape(n, _C_IN, ho, 2 * w_in)
    j = jax.lax.broadcasted_iota(jnp.int32, (w_in, _KW * wo), 0)
    col = jax.lax.broadcasted_iota(jnp.int32, (w_in, _KW * wo), 1)
    sel = 2 * (col % wo) + col // wo - 1
    t = (j == sel).astype(jnp.bfloat16)

    kern = functools.partial(_im2col_stats_kernel, ho=ho, wo=wo,
                             out_depth=out_depth)
    return pl.pallas_call(
        kern,
        out_shape=(jax.ShapeDtypeStruct((n, _K, ho, wo), jnp.bfloat16),
                   jax.ShapeDtypeStruct((n, c_total, 1), jnp.float32),
                   jax.ShapeDtypeStruct((n, c_total, 1), jnp.float32)),
        grid_spec=pltpu.PrefetchScalarGridSpec(
            num_scalar_prefetch=0,
            grid=(n,),
            in_specs=[pl.BlockSpec((1, _C_IN, ho, 2 * w_in),
                                   lambda i: (i, 0, 0, 0)),
                      pl.BlockSpec((w_in, _KW * wo), lambda i: (0, 0)),
                      pl.BlockSpec((c_total * _G, 2 * 128),
                                   lambda i: (0, 0))],
            out_specs=[pl.BlockSpec((1, _K, ho, wo), lambda i: (i, 0, 0, 0)),
                       pl.BlockSpec((1, c_total, 1), lambda i: (i, 0, 0)),
                       pl.BlockSpec((1, c_total, 1), lambda i: (i, 0, 0))]),
        compiler_params=pltpu.CompilerParams(
            dimension_semantics=("parallel",),
            vmem_limit_bytes=64 * 1024 * 1024),
    )(x4, t, l_mat)


def _conv_groups(p4, l_ref, rb, wo):
    l = l_ref[...]
    zpad = jnp.zeros((2 * 128 - _K * _G, wo), jnp.bfloat16)
    outs = []
    for g in range(rb // _G):
        rhs = p4[:, g * _G:(g + 1) * _G, :].reshape(_K * _G, wo)
        rhs = jnp.concatenate([rhs, zpad], axis=0)
        stk = jnp.dot(l, rhs, preferred_element_type=jnp.float32)
        outs.append(stk.reshape(16, _G, wo))
    return jnp.concatenate(outs, axis=1)


def _pool3(p4):
    pools = []
    for c in range(_C_IN):
        b = c * _KH * _KW
        m0 = jnp.maximum(p4[b + 4], p4[b + 5])
        m1 = jnp.maximum(p4[b + 7], p4[b + 8])
        pools.append(jnp.maximum(m0, m1)[None])
    return jnp.concatenate(pools, axis=0).astype(jnp.float32)


def _out_kernel(p_ref, l_ref, scale_ref, shift_ref, alpha_ref, out_ref,
                *, rb, wo, out_depth):
    p4 = p_ref[0]
    conv = _conv_groups(p4, l_ref, rb, wo)
    raw = jnp.concatenate([conv[:out_depth], _pool3(p4)], axis=0)
    y = raw * scale_ref[...][:, :, None] + shift_ref[...][:, :, None]
    out_ref[0] = jnp.where(y >= 0.0, y, alpha_ref[...][:, :, None] * y)


def _initial_block(x, conv_w, gamma, beta, alpha):
    N, c_in, H, W = x.shape
    assert c_in == _C_IN
    out_depth = conv_w.shape[0]
    c_total = out_depth + _C_IN
    Ho, Wo = H // 2, W // 2
    M = N * Ho * Wo

    rb = _G
    for cand_rb in (256, 128, 64, 32, 16):
        if Ho % cand_rb == 0:
            rb = cand_rb
            break
    assert Wo % 128 == 0 and Ho % rb == 0, "unsupported shape"
    tpi = Ho // rb
    n_tiles = N * tpi

    w_mat = jnp.pad(conv_w.reshape(out_depth, _K), ((0, c_total - out_depth),
                                                    (0, 0)))
    l_mat = jnp.kron(w_mat, jnp.eye(_G, dtype=w_mat.dtype))
    l_mat = jnp.pad(l_mat, ((0, 0), (0, 2 * 128 - _K * _G)))
    l_mat = l_mat.astype(jnp.bfloat16)

    patches, psum, psq = _im2col_pallas(x, l_mat, Ho, Wo, out_depth, c_total)

    ssum = jnp.sum(psum[:, :, 0], axis=0)
    ssq = jnp.sum(psq[:, :, 0], axis=0)
    mean = ssum / M
    var = jnp.maximum(ssq / M - mean * mean, 0.0)
    inv = jax.lax.rsqrt(var + _EPS)
    g = gamma.astype(jnp.float32)
    scale = (g * inv).reshape(c_total, 1)
    shift = (beta.astype(jnp.float32) - mean * g * inv).reshape(c_total, 1)
    alpha_col = jnp.broadcast_to(jnp.asarray(alpha, jnp.float32), (c_total, 1))

    kern_b = functools.partial(_out_kernel, rb=rb, wo=Wo, out_depth=out_depth)
    y = pl.pallas_call(
        kern_b,
        out_shape=jax.ShapeDtypeStruct((N, c_total, Ho, Wo), jnp.float32),
        grid_spec=pltpu.PrefetchScalarGridSpec(
            num_scalar_prefetch=0,
            grid=(n_tiles,),
            in_specs=[pl.BlockSpec((1, _K, rb, Wo),
                                   lambda i: (i // tpi, 0, i % tpi, 0)),
                      pl.BlockSpec((c_total * _G, 2 * 128),
                                   lambda i: (0, 0)),
                      pl.BlockSpec((c_total, 1), lambda i: (0, 0)),
                      pl.BlockSpec((c_total, 1), lambda i: (0, 0)),
                      pl.BlockSpec((c_total, 1), lambda i: (0, 0))],
            out_specs=pl.BlockSpec((1, c_total, rb, Wo),
                                   lambda i: (i // tpi, 0, i % tpi, 0))),
        compiler_params=pltpu.CompilerParams(
            dimension_semantics=("parallel",),
            vmem_limit_bytes=64 * 1024 * 1024),
    )(patches, l_mat, scale, shift, alpha_col)

    return y


def kernel(x, conv_w, gamma, beta, alpha):
    return _initial_block(x, conv_w, gamma, beta, alpha)
